# baseline (speedup 1.0000x reference)
import functools

import jax
import jax.numpy as jnp
from jax import lax
from jax.experimental import pallas as pl
from jax.experimental.pallas import tpu as pltpu
from jax.experimental.pallas import tpu_sc as plsc

_H = 128
_NB = 320000
_NA = 10000
_NA_PAD = 10240
_NMOL = 256
_MAXNB = 10
_PAD0 = 479
_G0 = 20480
_T = _G0 + _NB
_NW = 32
_NSTREAM = 5
_SB = 80
_CHUNK = (_NSTREAM * _SB) // _MAXNB
_BT = 2560
_DEPTH = 6


def _sc_gather_sum(nchunks: int, nrows_out: int):
    mesh = plsc.VectorSubcoreMesh(
        core_axis_name="c", subcore_axis_name="s", num_cores=2, num_subcores=16
    )

    assert nchunks % 2 == 0

    @functools.partial(
        pl.kernel,
        mesh=mesh,
        out_type=jax.ShapeDtypeStruct((nrows_out, _H), jnp.float32),
        scratch_types=[
            pltpu.VMEM((_NSTREAM * _SB,), jnp.int32),
            pltpu.VMEM((_NSTREAM * _SB,), jnp.int32),
            pltpu.VMEM((2, _NSTREAM * _SB, _H), jnp.float32),
            pltpu.VMEM((_CHUNK, _H), jnp.float32),
            pltpu.SemaphoreType.DMA,
            pltpu.SemaphoreType.DMA,
            pltpu.SemaphoreType.DMA,
            pltpu.SemaphoreType.DMA,
        ],
    )
    def gsum(table, idx, out, idx_v0, idx_v1, rows_v, acc_v,
             sg0, sg1, si0, si1):
        wid = lax.axis_index("s") * 2 + lax.axis_index("c")
        idxs = (idx_v0, idx_v1)
        sg = (sg0, sg1)
        si = (si0, si1)

        def fire_idx(c, slot):
            pltpu.async_copy(idx.at[wid, c], idxs[slot], si[slot])

        def wait_idx(slot):
            pltpu.make_async_copy(
                idx.at[wid, 0], idxs[slot], si[slot]
            ).wait()

            @plsc.parallel_loop(0, _NSTREAM * _SB // 16)
            def _(v):
                sl = pl.ds(v * 16, 16)
                idxs[slot][sl] = idxs[slot][sl] + _PAD0

        def fire_g(slot):
            for j in range(_NSTREAM):
                pltpu.async_copy(
                    table.at[idxs[slot].at[pl.ds(j * _SB, _SB)]],
                    rows_v.at[slot, pl.ds(j * _SB, _SB)],
                    sg[slot],
                )

        def wait_g(slot):
            for j in range(_NSTREAM):
                pltpu.make_async_copy(
                    table.at[idxs[slot].at[pl.ds(j * _SB, _SB)]],
                    rows_v.at[slot, pl.ds(j * _SB, _SB)],
                    sg[slot],
                ).wait()

        def compute(slot, c):
            @plsc.parallel_loop(0, _CHUNK, unroll=2)
            def row_body(b):
                base = b * _MAXNB
                for d in range(_H // 16):
                    s = rows_v[slot, base, pl.ds(d * 16, 16)]
                    for k in range(1, _MAXNB):
                        s = s + rows_v[slot, base + k, pl.ds(d * 16, 16)]
                    acc_v[b, pl.ds(d * 16, 16)] = s
            pltpu.sync_copy(
                acc_v, out.at[pl.ds((wid * nchunks + c) * _CHUNK, _CHUNK)]
            )

        def step(c, slot):
            wait_g(slot)

            @pl.when(c + 2 < nchunks)
            def _():
                fire_idx(c + 2, slot)

            @pl.when(c + 1 < nchunks)
            def _():
                wait_idx(1 - slot)
                fire_g(1 - slot)

            compute(slot, c)

        fire_idx(0, 0)
        fire_idx(1, 1)
        wait_idx(0)
        fire_g(0)

        def body2(cc, carry):
            step(cc * 2, 0)
            step(cc * 2 + 1, 1)
            return carry

        lax.fori_loop(0, nchunks // 2, body2, 0)

    return gsum


_DN = (((1,), (1,)), ((), ()))


def _dot_t(x, w, precision=lax.Precision.DEFAULT):
    return lax.dot_general(
        x, w, dimension_numbers=_DN,
        preferred_element_type=jnp.float32,
        precision=precision,
    )


_NTREEB = _G0 // _BT


def _m0_body(tree_ref, fb_ref, wi_ref, mout_ref):
    i = pl.program_id(0)

    @pl.when(i < _NTREEB)
    def _():
        mout_ref[...] = tree_ref[...]

    @pl.when(i >= _NTREEB)
    def _():
        mout_ref[...] = jnp.maximum(_dot_t(fb_ref[...], wi_ref[...]), 0.0)


def _m0_call(tree_pad, fbonds, W_i):
    grid = _T // _BT
    return pl.pallas_call(
        _m0_body,
        grid=(grid,),
        in_specs=[
            pl.BlockSpec((_BT, _H), lambda i: (jnp.minimum(i, _NTREEB - 1), 0)),
            pl.BlockSpec((_BT, 40), lambda i: (jnp.maximum(i - _NTREEB, 0), 0)),
            pl.BlockSpec((_H, 40), lambda i: (0, 0)),
        ],
        out_specs=pl.BlockSpec((_BT, _H), lambda i: (i, 0)),
        out_shape=jax.ShapeDtypeStruct((_T, _H), jnp.float32),
        compiler_params=pltpu.CompilerParams(
            dimension_semantics=("arbitrary",)
        ),
    )(tree_pad, fbonds, W_i)


def _iter_body(nei_ref, fb_ref, wi_ref, wh_ref, mprev_ref, mout_ref):
    del mprev_ref
    bi = _dot_t(fb_ref[...], wi_ref[...])
    mout_ref[...] = jnp.maximum(
        bi + _dot_t(nei_ref[...], wh_ref[...]), 0.0
    )


def _iter_call(nei, fbonds, W_i, W_h, M_prev):
    grid = _NB // _BT
    return pl.pallas_call(
        _iter_body,
        grid=(grid,),
        in_specs=[
            pl.BlockSpec((_BT, _H), lambda i: (i, 0)),
            pl.BlockSpec((_BT, 40), lambda i: (i, 0)),
            pl.BlockSpec((_H, 40), lambda i: (0, 0)),
            pl.BlockSpec((_H, _H), lambda i: (0, 0)),
            pl.BlockSpec(memory_space=pltpu.MemorySpace.HBM),
        ],
        out_specs=pl.BlockSpec((_BT, _H), lambda i: (i + _G0 // _BT, 0)),
        out_shape=jax.ShapeDtypeStruct((_T, _H), jnp.float32),
        input_output_aliases={4: 0},
        compiler_params=pltpu.CompilerParams(
            dimension_semantics=("parallel",)
        ),
    )(nei, fbonds, W_i, W_h, M_prev)


_ABT = 2000


def _final_body(fa_ref, nei_ref, w1_ref, w2_ref, bo_ref, seg_ref,
                out_ref, acc_h, acc_c):
    i = pl.program_id(0)

    @pl.when(i == 0)
    def _():
        acc_h[...] = jnp.zeros_like(acc_h)
        acc_c[...] = jnp.zeros_like(acc_c)

    h = jnp.maximum(
        _dot_t(fa_ref[...], w1_ref[...])
        + _dot_t(nei_ref[...], w2_ref[...])
        + bo_ref[...],
        0.0,
    )
    seg = seg_ref[0]
    oht = (
        lax.broadcasted_iota(jnp.int32, (_NMOL, _ABT), 0) == seg
    ).astype(jnp.float32)
    acc_h[...] += lax.dot_general(
        oht, h, dimension_numbers=(((1,), (0,)), ((), ())),
        preferred_element_type=jnp.float32,
        precision=lax.Precision.HIGHEST,
    )
    acc_c[...] += jnp.broadcast_to(
        jnp.sum(oht, axis=1, keepdims=True), (_NMOL, _H)
    )

    @pl.when(i == pl.num_programs(0) - 1)
    def _():
        out_ref[...] = acc_h[...] / jnp.maximum(acc_c[...], 1.0)


def _final_call(fatoms, nei_a, W_o, b_o, seg_ids):
    grid = _NA // _ABT
    w1 = W_o[:, :35]
    w2 = W_o[:, 35:]
    seg3 = seg_ids.reshape(grid, 1, _ABT)
    return pl.pallas_call(
        _final_body,
        grid=(grid,),
        in_specs=[
            pl.BlockSpec((_ABT, 35), lambda i: (i, 0)),
            pl.BlockSpec((_ABT, _H), lambda i: (i, 0)),
            pl.BlockSpec((_H, 35), lambda i: (0, 0)),
            pl.BlockSpec((_H, _H), lambda i: (0, 0)),
            pl.BlockSpec((1, _H), lambda i: (0, 0)),
            pl.BlockSpec((1, 1, _ABT), lambda i: (i, 0, 0)),
        ],
        out_specs=pl.BlockSpec((_NMOL, _H), lambda i: (0, 0)),
        out_shape=jax.ShapeDtypeStruct((_NMOL, _H), jnp.float32),
        scratch_shapes=[
            pltpu.VMEM((_NMOL, _H), jnp.float32),
            pltpu.VMEM((_NMOL, _H), jnp.float32),
        ],
        compiler_params=pltpu.CompilerParams(
            dimension_semantics=("arbitrary",)
        ),
    )(fatoms, nei_a, w1, w2, b_o.reshape(1, _H), seg3)


def kernel(fatoms, fbonds, tree_message, W_i, W_h, W_o, b_o, agraph, bgraph,
           seg_ids):
    bg = bgraph.astype(jnp.int32).reshape(
        _NW, _NB // (_NW * _CHUNK), _NSTREAM * _SB
    )
    ag = jnp.concatenate(
        [
            agraph.astype(jnp.int32),
            jnp.zeros((_NA_PAD - _NA, _MAXNB), jnp.int32),
        ],
        axis=0,
    ).reshape(_NW, _NA_PAD // (_NW * _CHUNK), _NSTREAM * _SB)

    tree_pad = jnp.pad(tree_message, ((_PAD0, 0), (0, 0)))
    M = _m0_call(tree_pad, fbonds, W_i)

    gsum_b = _sc_gather_sum(_NB // (_NW * _CHUNK), _NB)
    for _ in range(_DEPTH - 1):
        nei = gsum_b(M, bg)
        M = _iter_call(nei, fbonds, W_i, W_h, M)

    gsum_a = _sc_gather_sum(_NA_PAD // (_NW * _CHUNK), _NA_PAD)
    nei_a = gsum_a(M, ag)[:_NA]

    return _final_call(fatoms, nei_a, W_o, b_o, seg_ids.astype(jnp.int32))


# speedup vs baseline: 1.2818x; 1.2818x over previous
import functools

import numpy as np

import jax
import jax.numpy as jnp
from jax import lax
from jax.experimental import pallas as pl
from jax.experimental.pallas import tpu as pltpu
from jax.experimental.pallas import tpu_sc as plsc

_H = 128
_NB = 320000
_NA = 10000
_NA_PAD = 10240
_NMOL = 256
_MAXNB = 10
_PAD0 = 479
_G0 = 20480
_T = _G0 + _NB
_NW = 32
_NSTREAM = 10
_SB = 80
_CHUNK = (_NSTREAM * _SB) // _MAXNB
_BT = 2560
_DEPTH = 6


def _sc_gather_sum(nchunks: int, nrows_out: int):
    mesh = plsc.VectorSubcoreMesh(
        core_axis_name="c", subcore_axis_name="s", num_cores=2, num_subcores=16
    )

    @functools.partial(
        pl.kernel,
        mesh=mesh,
        compiler_params=pltpu.CompilerParams(
            needs_layout_passes=False, use_tc_tiling_on_sc=False
        ),
        out_type=jax.ShapeDtypeStruct((nrows_out, _H), jnp.float32),
        scratch_types=[
            pltpu.VMEM((_NSTREAM * _SB,), jnp.int32),
            pltpu.VMEM((_NSTREAM * _SB,), jnp.int32),
            pltpu.VMEM((2, _NSTREAM * _SB, _H // 2), jnp.uint32),
            pltpu.VMEM((_CHUNK, _H), jnp.float32),
            pltpu.SemaphoreType.DMA,
            pltpu.SemaphoreType.DMA,
            pltpu.SemaphoreType.DMA,
            pltpu.SemaphoreType.DMA,
        ],
    )
    def gsum(table, idx, out, idx_v0, idx_v1, rows_v, acc_v,
             sg0, sg1, si0, si1):
        wid = lax.axis_index("s") * 2 + lax.axis_index("c")
        idxs = (idx_v0, idx_v1)
        sg = (sg0, sg1)
        si = (si0, si1)

        def fire_idx(c, slot):
            pltpu.async_copy(idx.at[wid, c], idxs[slot], si[slot])

        def wait_idx(slot):
            pltpu.make_async_copy(
                idx.at[wid, 0], idxs[slot], si[slot]
            ).wait()

        def fire_g(slot):
            for j in range(_NSTREAM):
                pltpu.async_copy(
                    table.at[idxs[slot].at[pl.ds(j * _SB, _SB)]],
                    rows_v.at[slot, pl.ds(j * _SB, _SB)],
                    sg[slot],
                )

        def wait_g(slot):
            for j in range(_NSTREAM):
                pltpu.make_async_copy(
                    table.at[idxs[slot].at[pl.ds(j * _SB, _SB)]],
                    rows_v.at[slot, pl.ds(j * _SB, _SB)],
                    sg[slot],
                ).wait()

        def compute(slot, c):
            @plsc.parallel_loop(0, _CHUNK, unroll=2)
            def row_body(b):
                base = b * _MAXNB
                for g in range(_H // 32):
                    lo = hi = None
                    for k in range(_MAXNB):
                        x = rows_v[slot, base + k, pl.ds(g * 16, 16)]
                        l = plsc.bitcast(
                            jnp.left_shift(x, jnp.uint32(16)), jnp.float32
                        )
                        h = plsc.bitcast(
                            jnp.bitwise_and(x, jnp.uint32(0xFFFF0000)),
                            jnp.float32,
                        )
                        lo = l if lo is None else lo + l
                        hi = h if hi is None else hi + h
                    acc_v[b, pl.ds(g * 16, 16)] = lo
                    acc_v[b, pl.ds(64 + g * 16, 16)] = hi
            pltpu.sync_copy(
                acc_v, out.at[pl.ds((wid * nchunks + c) * _CHUNK, _CHUNK)]
            )

        def step(c, slot):
            wait_g(slot)

            @pl.when(c + 2 < nchunks)
            def _():
                fire_idx(c + 2, slot)

            @pl.when(c + 1 < nchunks)
            def _():
                wait_idx(1 - slot)
                fire_g(1 - slot)

            compute(slot, c)

        fire_idx(0, 0)
        fire_idx(1, 1)
        wait_idx(0)
        fire_g(0)

        def body2(cc, carry):
            step(cc * 2, 0)
            step(cc * 2 + 1, 1)
            return carry

        lax.fori_loop(0, nchunks // 2, body2, 0)
        if nchunks % 2:
            step(jnp.int32(nchunks - 1), 0)

    return gsum


_DN = (((1,), (1,)), ((), ()))


def _dot_t(x, w, precision=lax.Precision.DEFAULT):
    return lax.dot_general(
        x, w, dimension_numbers=_DN,
        preferred_element_type=jnp.float32,
        precision=precision,
    )


_NTREEB = _G0 // _BT


def _pack_cols(y):
    a = lax.bitcast_convert_type(
        y[:, : _H // 2].astype(jnp.bfloat16), jnp.uint16
    ).astype(jnp.uint32)
    b = lax.bitcast_convert_type(
        y[:, _H // 2:].astype(jnp.bfloat16), jnp.uint16
    ).astype(jnp.uint32)
    return a | (b << jnp.uint32(16))


def _m0_body(tree_ref, fb_ref, wi_ref, mout_ref):
    i = pl.program_id(0)

    @pl.when(i < _NTREEB)
    def _():
        mout_ref[...] = _pack_cols(tree_ref[...])

    @pl.when(i >= _NTREEB)
    def _():
        mout_ref[...] = _pack_cols(
            jnp.maximum(_dot_t(fb_ref[...], wi_ref[...]), 0.0)
        )


def _m0_call(tree_pad, fbonds, W_i):
    grid = _T // _BT
    return pl.pallas_call(
        _m0_body,
        grid=(grid,),
        in_specs=[
            pl.BlockSpec((_BT, _H), lambda i: (jnp.minimum(i, _NTREEB - 1), 0)),
            pl.BlockSpec((_BT, 40), lambda i: (jnp.maximum(i - _NTREEB, 0), 0)),
            pl.BlockSpec((_H, 40), lambda i: (0, 0)),
        ],
        out_specs=pl.BlockSpec((_BT, _H // 2), lambda i: (i, 0)),
        out_shape=jax.ShapeDtypeStruct((_T, _H // 2), jnp.uint32),
        compiler_params=pltpu.CompilerParams(
            dimension_semantics=("arbitrary",)
        ),
    )(tree_pad, fbonds, W_i)


def _iter_body(nei_ref, fb_ref, wi_ref, wh_ref, mprev_ref, mout_ref):
    del mprev_ref
    bi = _dot_t(fb_ref[...], wi_ref[...])
    mout_ref[...] = _pack_cols(
        jnp.maximum(bi + _dot_t(nei_ref[...], wh_ref[...]), 0.0)
    )


def _iter_call(nei, fbonds, W_i, W_h, M_prev):
    grid = _NB // _BT
    return pl.pallas_call(
        _iter_body,
        grid=(grid,),
        in_specs=[
            pl.BlockSpec((_BT, _H), lambda i: (i, 0)),
            pl.BlockSpec((_BT, 40), lambda i: (i, 0)),
            pl.BlockSpec((_H, 40), lambda i: (0, 0)),
            pl.BlockSpec((_H, _H), lambda i: (0, 0)),
            pl.BlockSpec(memory_space=pltpu.MemorySpace.HBM),
        ],
        out_specs=pl.BlockSpec((_BT, _H // 2), lambda i: (i + _G0 // _BT, 0)),
        out_shape=jax.ShapeDtypeStruct((_T, _H // 2), jnp.uint32),
        input_output_aliases={4: 0},
        compiler_params=pltpu.CompilerParams(
            dimension_semantics=("parallel",)
        ),
    )(nei, fbonds, W_i, W_h, M_prev)


_ABT = 2000


def _final_body(fa_ref, nei_ref, w1_ref, w2_ref, bo_ref, seg_ref,
                out_ref, acc_h, acc_c):
    i = pl.program_id(0)

    @pl.when(i == 0)
    def _():
        acc_h[...] = jnp.zeros_like(acc_h)
        acc_c[...] = jnp.zeros_like(acc_c)

    h = jnp.maximum(
        _dot_t(fa_ref[...], w1_ref[...])
        + _dot_t(nei_ref[...], w2_ref[...])
        + bo_ref[...],
        0.0,
    )
    seg = seg_ref[0]
    oht = (
        lax.broadcasted_iota(jnp.int32, (_NMOL, _ABT), 0) == seg
    ).astype(jnp.float32)
    acc_h[...] += lax.dot_general(
        oht, h, dimension_numbers=(((1,), (0,)), ((), ())),
        preferred_element_type=jnp.float32,
        precision=lax.Precision.HIGHEST,
    )
    acc_c[...] += jnp.broadcast_to(
        jnp.sum(oht, axis=1, keepdims=True), (_NMOL, _H)
    )

    @pl.when(i == pl.num_programs(0) - 1)
    def _():
        out_ref[...] = acc_h[...] / jnp.maximum(acc_c[...], 1.0)


def _final_call(fatoms, nei_a, W_o, b_o, seg_ids):
    grid = _NA // _ABT
    w1 = W_o[:, :35]
    w2 = W_o[:, 35:]
    seg3 = seg_ids.reshape(grid, 1, _ABT)
    return pl.pallas_call(
        _final_body,
        grid=(grid,),
        in_specs=[
            pl.BlockSpec((_ABT, 35), lambda i: (i, 0)),
            pl.BlockSpec((_ABT, _H), lambda i: (i, 0)),
            pl.BlockSpec((_H, 35), lambda i: (0, 0)),
            pl.BlockSpec((_H, _H), lambda i: (0, 0)),
            pl.BlockSpec((1, _H), lambda i: (0, 0)),
            pl.BlockSpec((1, 1, _ABT), lambda i: (i, 0, 0)),
        ],
        out_specs=pl.BlockSpec((_NMOL, _H), lambda i: (0, 0)),
        out_shape=jax.ShapeDtypeStruct((_NMOL, _H), jnp.float32),
        scratch_shapes=[
            pltpu.VMEM((_NMOL, _H), jnp.float32),
            pltpu.VMEM((_NMOL, _H), jnp.float32),
        ],
        compiler_params=pltpu.CompilerParams(
            dimension_semantics=("arbitrary",)
        ),
    )(fatoms, nei_a, w1, w2, b_o.reshape(1, _H), seg3)


def kernel(fatoms, fbonds, tree_message, W_i, W_h, W_o, b_o, agraph, bgraph,
           seg_ids):
    bg = (bgraph.astype(jnp.int32) + _PAD0).reshape(
        _NW, _NB // (_NW * _CHUNK), _NSTREAM * _SB
    )
    ag = (
        jnp.concatenate(
            [
                agraph.astype(jnp.int32),
                jnp.zeros((_NA_PAD - _NA, _MAXNB), jnp.int32),
            ],
            axis=0,
        )
        + _PAD0
    ).reshape(_NW, _NA_PAD // (_NW * _CHUNK), _NSTREAM * _SB)

    tree_pad = jnp.pad(tree_message, ((_PAD0, 0), (0, 0)))
    M = _m0_call(tree_pad, fbonds, W_i)

    gsum_b = _sc_gather_sum(_NB // (_NW * _CHUNK), _NB)
    for _ in range(_DEPTH - 1):
        nei = gsum_b(M, bg)
        M = _iter_call(nei, fbonds, W_i, W_h, M)

    gsum_a = _sc_gather_sum(_NA_PAD // (_NW * _CHUNK), _NA_PAD)
    nei_a = gsum_a(M, ag)[:_NA]

    return _final_call(fatoms, nei_a, W_o, b_o, seg_ids.astype(jnp.int32))


# speedup vs baseline: 1.2830x; 1.0010x over previous
import functools

import jax
import jax.numpy as jnp
from jax import lax
from jax.experimental import pallas as pl
from jax.experimental.pallas import tpu as pltpu
from jax.experimental.pallas import tpu_sc as plsc

_H = 128
_NB = 320000
_NA = 10000
_NA_PAD = 10240
_NMOL = 256
_MAXNB = 10
_PAD0 = 479
_G0 = 20480
_T = _G0 + _NB
_NW = 32
_NSTREAM = 10
_SB = 80
_CHUNK = (_NSTREAM * _SB) // _MAXNB
_BT = 2560
_DEPTH = 6


def _sc_gather_sum(nchunks: int, nrows_out: int):
    mesh = plsc.VectorSubcoreMesh(
        core_axis_name="c", subcore_axis_name="s", num_cores=2, num_subcores=16
    )

    @functools.partial(
        pl.kernel,
        mesh=mesh,
        compiler_params=pltpu.CompilerParams(
            needs_layout_passes=False, use_tc_tiling_on_sc=False
        ),
        out_type=jax.ShapeDtypeStruct((nrows_out, _H), jnp.float32),
        scratch_types=[
            pltpu.VMEM((_NSTREAM * _SB,), jnp.int32),
            pltpu.VMEM((_NSTREAM * _SB,), jnp.int32),
            pltpu.VMEM((2, _NSTREAM * _SB, _H // 2), jnp.uint32),
            pltpu.VMEM((_CHUNK, _H), jnp.float32),
            pltpu.SemaphoreType.DMA,
            pltpu.SemaphoreType.DMA,
            pltpu.SemaphoreType.DMA,
            pltpu.SemaphoreType.DMA,
        ],
    )
    def gsum(table, idx, out, idx_v0, idx_v1, rows_v, acc_v,
             sg0, sg1, si0, si1):
        wid = lax.axis_index("s") * 2 + lax.axis_index("c")
        idxs = (idx_v0, idx_v1)
        sg = (sg0, sg1)
        si = (si0, si1)

        def fire_idx(c, slot):
            pltpu.async_copy(idx.at[wid, c], idxs[slot], si[slot])

        def wait_idx(slot):
            pltpu.make_async_copy(
                idx.at[wid, 0], idxs[slot], si[slot]
            ).wait()

        def fire_g(slot):
            for j in range(_NSTREAM):
                pltpu.async_copy(
                    table.at[idxs[slot].at[pl.ds(j * _SB, _SB)]],
                    rows_v.at[slot, pl.ds(j * _SB, _SB)],
                    sg[slot],
                )

        def wait_g(slot):
            for j in range(_NSTREAM):
                pltpu.make_async_copy(
                    table.at[idxs[slot].at[pl.ds(j * _SB, _SB)]],
                    rows_v.at[slot, pl.ds(j * _SB, _SB)],
                    sg[slot],
                ).wait()

        def compute(slot, c):
            @plsc.parallel_loop(0, _CHUNK, unroll=2)
            def row_body(b):
                base = b * _MAXNB
                for g in range(_H // 32):
                    lo = hi = None
                    for k in range(_MAXNB):
                        x = rows_v[slot, base + k, pl.ds(g * 16, 16)]
                        l = plsc.bitcast(
                            jnp.left_shift(x, jnp.uint32(16)), jnp.float32
                        )
                        h = plsc.bitcast(
                            jnp.bitwise_and(x, jnp.uint32(0xFFFF0000)),
                            jnp.float32,
                        )
                        lo = l if lo is None else lo + l
                        hi = h if hi is None else hi + h
                    acc_v[b, pl.ds(g * 16, 16)] = lo
                    acc_v[b, pl.ds(64 + g * 16, 16)] = hi
            pltpu.sync_copy(
                acc_v, out.at[pl.ds((wid * nchunks + c) * _CHUNK, _CHUNK)]
            )

        def step(c, slot):
            wait_g(slot)

            @pl.when(c + 2 < nchunks)
            def _():
                fire_idx(c + 2, slot)

            @pl.when(c + 1 < nchunks)
            def _():
                wait_idx(1 - slot)
                fire_g(1 - slot)

            compute(slot, c)

        fire_idx(0, 0)
        fire_idx(1, 1)
        wait_idx(0)
        fire_g(0)

        def body2(cc, carry):
            step(cc * 2, 0)
            step(cc * 2 + 1, 1)
            return carry

        lax.fori_loop(0, nchunks // 2, body2, 0)
        if nchunks % 2:
            step(jnp.int32(nchunks - 1), 0)

    return gsum


_DN = (((1,), (1,)), ((), ()))


def _dot_t(x, w, precision=lax.Precision.DEFAULT):
    return lax.dot_general(
        x, w, dimension_numbers=_DN,
        preferred_element_type=jnp.float32,
        precision=precision,
    )


_NTREEB = _G0 // _BT


def _pack_cols(y):
    a = lax.bitcast_convert_type(
        y[:, : _H // 2].astype(jnp.bfloat16), jnp.uint16
    ).astype(jnp.uint32)
    b = lax.bitcast_convert_type(
        y[:, _H // 2:].astype(jnp.bfloat16), jnp.uint16
    ).astype(jnp.uint32)
    return a | (b << jnp.uint32(16))


def _m0_body(tree_ref, fb_ref, wi_ref, mout_ref):
    i = pl.program_id(0)

    @pl.when(i < _NTREEB)
    def _():
        mout_ref[...] = _pack_cols(tree_ref[...])

    @pl.when(i >= _NTREEB)
    def _():
        mout_ref[...] = _pack_cols(
            jnp.maximum(_dot_t(fb_ref[...], wi_ref[...]), 0.0)
        )


def _m0_call(tree_pad, fbonds, W_i):
    grid = _T // _BT
    return pl.pallas_call(
        _m0_body,
        grid=(grid,),
        in_specs=[
            pl.BlockSpec((_BT, _H), lambda i: (jnp.minimum(i, _NTREEB - 1), 0)),
            pl.BlockSpec((_BT, 40), lambda i: (jnp.maximum(i - _NTREEB, 0), 0)),
            pl.BlockSpec((_H, 40), lambda i: (0, 0)),
        ],
        out_specs=pl.BlockSpec((_BT, _H // 2), lambda i: (i, 0)),
        out_shape=jax.ShapeDtypeStruct((_T, _H // 2), jnp.uint32),
        compiler_params=pltpu.CompilerParams(
            dimension_semantics=("arbitrary",)
        ),
    )(tree_pad, fbonds, W_i)


def _iter_body(nei_ref, fb_ref, wi_ref, wh_ref, mprev_ref, mout_ref):
    del mprev_ref
    bi = _dot_t(fb_ref[...], wi_ref[...])
    mout_ref[...] = _pack_cols(
        jnp.maximum(bi + _dot_t(nei_ref[...], wh_ref[...]), 0.0)
    )


def _iter_call(nei, fbonds, W_i, W_h, M_prev):
    grid = _NB // _BT
    return pl.pallas_call(
        _iter_body,
        grid=(grid,),
        in_specs=[
            pl.BlockSpec((_BT, _H), lambda i: (i, 0)),
            pl.BlockSpec((_BT, 40), lambda i: (i, 0)),
            pl.BlockSpec((_H, 40), lambda i: (0, 0)),
            pl.BlockSpec((_H, _H), lambda i: (0, 0)),
            pl.BlockSpec(memory_space=pltpu.MemorySpace.HBM),
        ],
        out_specs=pl.BlockSpec((_BT, _H // 2), lambda i: (i + _G0 // _BT, 0)),
        out_shape=jax.ShapeDtypeStruct((_T, _H // 2), jnp.uint32),
        input_output_aliases={4: 0},
        compiler_params=pltpu.CompilerParams(
            dimension_semantics=("parallel",)
        ),
    )(nei, fbonds, W_i, W_h, M_prev)


_ABT = 2000


def _final_body(fa_ref, nei_ref, w1_ref, w2_ref, bo_ref, seg_ref,
                out_ref, acc_h, acc_c):
    i = pl.program_id(0)

    @pl.when(i == 0)
    def _():
        acc_h[...] = jnp.zeros_like(acc_h)
        acc_c[...] = jnp.zeros_like(acc_c)

    h = jnp.maximum(
        _dot_t(fa_ref[...], w1_ref[...])
        + _dot_t(nei_ref[...], w2_ref[...])
        + bo_ref[...],
        0.0,
    )
    seg = seg_ref[0]
    oht = (
        lax.broadcasted_iota(jnp.int32, (_NMOL, _ABT), 0) == seg
    ).astype(jnp.float32)
    acc_h[...] += lax.dot_general(
        oht, h, dimension_numbers=(((1,), (0,)), ((), ())),
        preferred_element_type=jnp.float32,
        precision=lax.Precision.HIGHEST,
    )
    acc_c[...] += jnp.broadcast_to(
        jnp.sum(oht, axis=1, keepdims=True), (_NMOL, _H)
    )

    @pl.when(i == pl.num_programs(0) - 1)
    def _():
        out_ref[...] = acc_h[...] / jnp.maximum(acc_c[...], 1.0)


def _final_call(fatoms, nei_a, W_o, b_o, seg_ids):
    grid = _NA // _ABT
    w1 = W_o[:, :35]
    w2 = W_o[:, 35:]
    seg3 = seg_ids.reshape(grid, 1, _ABT)
    return pl.pallas_call(
        _final_body,
        grid=(grid,),
        in_specs=[
            pl.BlockSpec((_ABT, 35), lambda i: (i, 0)),
            pl.BlockSpec((_ABT, _H), lambda i: (i, 0)),
            pl.BlockSpec((_H, 35), lambda i: (0, 0)),
            pl.BlockSpec((_H, _H), lambda i: (0, 0)),
            pl.BlockSpec((1, _H), lambda i: (0, 0)),
            pl.BlockSpec((1, 1, _ABT), lambda i: (i, 0, 0)),
        ],
        out_specs=pl.BlockSpec((_NMOL, _H), lambda i: (0, 0)),
        out_shape=jax.ShapeDtypeStruct((_NMOL, _H), jnp.float32),
        scratch_shapes=[
            pltpu.VMEM((_NMOL, _H), jnp.float32),
            pltpu.VMEM((_NMOL, _H), jnp.float32),
        ],
        compiler_params=pltpu.CompilerParams(
            dimension_semantics=("arbitrary",)
        ),
    )(fatoms, nei_a, w1, w2, b_o.reshape(1, _H), seg3)


def kernel(fatoms, fbonds, tree_message, W_i, W_h, W_o, b_o, agraph, bgraph,
           seg_ids):
    bg = (bgraph.astype(jnp.int32) + _PAD0).reshape(
        _NW, _NB // (_NW * _CHUNK), _NSTREAM * _SB
    )
    ag = (
        jnp.concatenate(
            [
                agraph.astype(jnp.int32),
                jnp.zeros((_NA_PAD - _NA, _MAXNB), jnp.int32),
            ],
            axis=0,
        )
        + _PAD0
    ).reshape(_NW, _NA_PAD // (_NW * _CHUNK), _NSTREAM * _SB)

    tree_pad = jnp.pad(tree_message, ((_PAD0, 0), (0, 0)))
    M = _m0_call(tree_pad, fbonds, W_i)

    gsum_b = _sc_gather_sum(_NB // (_NW * _CHUNK), _NB)
    for _ in range(_DEPTH - 1):
        nei = gsum_b(M, bg)
        M = _iter_call(nei, fbonds, W_i, W_h, M)

    gsum_a = _sc_gather_sum(_NA_PAD // (_NW * _CHUNK), _NA_PAD)
    nei_a = gsum_a(M, ag)[:_NA]

    return _final_call(fatoms, nei_a, W_o, b_o, seg_ids.astype(jnp.int32))
